# Initial kernel scaffold; baseline (speedup 1.0000x reference)
import jax
import jax.numpy as jnp
from jax.experimental import pallas as pl


def kernel(z, edge_index):
    raise NotImplementedError("write your pallas kernel here")


# speedup vs baseline: 1.0959x; 1.0959x over previous
import functools

import jax
import jax.numpy as jnp
from jax import lax
from jax.experimental import pallas as pl
from jax.experimental.pallas import tpu as pltpu
from jax.experimental.pallas import tpu_sc as plsc

NC = 2
NS = 16
L = 16
NW = NC * NS

E = 320000
D = 128
PER_W = E // NW
CHUNK = 80
N_ITERS = PER_W // CHUNK


def _sc_body(z_hbm, src_hbm, dst_hbm, out_hbm,
             sidx_v, didx_v, srows_v, drows_v, out_v, sem_s, sem_d):
    wid = lax.axis_index("s") * NC + lax.axis_index("c")
    lane = lax.iota(jnp.int32, L)

    def iter_body(i, carry):
        base = wid * PER_W + i * CHUNK
        pltpu.sync_copy(src_hbm.at[pl.ds(base, CHUNK)], sidx_v)
        pltpu.sync_copy(dst_hbm.at[pl.ds(base, CHUNK)], didx_v)
        cp_s = pltpu.async_copy(z_hbm.at[sidx_v], srows_v,
                                sem_s)
        cp_d = pltpu.async_copy(z_hbm.at[didx_v], drows_v,
                                sem_d)
        cp_s.wait()
        cp_d.wait()

        def group_body(g, carry2):
            eids = g * L + lane

            def d_blk(j, acc):
                for u in range(8):
                    dv = jnp.full((L,), j * 8 + u, dtype=jnp.int32)
                    s = plsc.load_gather(srows_v, [eids, dv])
                    t = plsc.load_gather(drows_v, [eids, dv])
                    acc = acc + s * t
                return acc

            acc = lax.fori_loop(0, D // 8, d_blk,
                                jnp.zeros((L,), jnp.float32))
            out_v[pl.ds(g * L, L)] = 1.0 / (1.0 + jnp.exp(-acc))
            return carry2

        lax.fori_loop(0, CHUNK // L, group_body, 0)
        pltpu.sync_copy(out_v, out_hbm.at[pl.ds(base, CHUNK)])
        return carry

    lax.fori_loop(0, N_ITERS, iter_body, 0)


@jax.jit
def _run(z, src, dst):
    mesh = plsc.VectorSubcoreMesh(
        core_axis_name="c", subcore_axis_name="s",
        num_cores=NC, num_subcores=NS)
    kfn = pl.kernel(
        _sc_body,
        out_type=jax.ShapeDtypeStruct((E,), jnp.float32),
        mesh=mesh,
        scratch_types=[
            pltpu.VMEM((CHUNK,), jnp.int32),
            pltpu.VMEM((CHUNK,), jnp.int32),
            pltpu.VMEM((CHUNK, D), jnp.float32),
            pltpu.VMEM((CHUNK, D), jnp.float32),
            pltpu.VMEM((CHUNK,), jnp.float32),
            pltpu.SemaphoreType.DMA,
            pltpu.SemaphoreType.DMA,
        ],
        compiler_params=pltpu.CompilerParams(needs_layout_passes=False),
    )
    return kfn(z, src, dst)


def kernel(z, edge_index):
    src = edge_index[0].astype(jnp.int32)
    dst = edge_index[1].astype(jnp.int32)
    return _run(z, src, dst)


# speedup vs baseline: 1.5321x; 1.3981x over previous
import functools

import jax
import jax.numpy as jnp
from jax import lax
from jax.experimental import pallas as pl
from jax.experimental.pallas import tpu as pltpu
from jax.experimental.pallas import tpu_sc as plsc

NC = 2
NS = 16
L = 16
NW = NC * NS

E = 320000
D = 128
PER_W = E // NW
CHUNK = 80
N_ITERS = PER_W // CHUNK
N_SLOTS = 5
OUT_W = 5 * N_SLOTS * CHUNK


def _sc_body(z_hbm, src_hbm, dst_hbm, out_hbm, sidx_v, didx_v,
             srows, drows, out_v, sem_i0, sem_i1, sem_s, sem_d):
    wid = lax.axis_index("s") * NC + lax.axis_index("c")
    lane = lax.iota(jnp.int32, L)
    base_w = wid * PER_W

    ci0 = pltpu.async_copy(src_hbm.at[pl.ds(base_w, PER_W)], sidx_v, sem_i0)
    ci1 = pltpu.async_copy(dst_hbm.at[pl.ds(base_w, PER_W)], didx_v, sem_i1)
    ci0.wait()
    ci1.wait()

    def issue(b, chunk):
        off = chunk * CHUNK
        pltpu.async_copy(
            z_hbm.at[sidx_v.at[pl.ds(off, CHUNK)]], srows[b], sem_s[b])
        pltpu.async_copy(
            z_hbm.at[didx_v.at[pl.ds(off, CHUNK)]], drows[b], sem_d[b])

    for b in range(N_SLOTS):
        issue(b, b)

    def compute(b, o, chunk):
        cbase = ((o % 5) * N_SLOTS + (chunk - o * N_SLOTS)) * CHUNK

        def group_body(g, carry):
            eids = g * L + lane

            def d_blk(j, accs):
                a0, a1, a2, a3 = accs
                prods = []
                for u in range(8):
                    dv = jnp.full((L,), 0, dtype=jnp.int32) + (j * 8 + u)
                    s = plsc.load_gather(srows[b], [eids, dv])
                    t = plsc.load_gather(drows[b], [eids, dv])
                    prods.append(s * t)
                a0 = a0 + (prods[0] + prods[1])
                a1 = a1 + (prods[2] + prods[3])
                a2 = a2 + (prods[4] + prods[5])
                a3 = a3 + (prods[6] + prods[7])
                return a0, a1, a2, a3

            z4 = jnp.zeros((L,), jnp.float32)
            a0, a1, a2, a3 = lax.fori_loop(0, D // 8, d_blk,
                                           (z4, z4, z4, z4))
            acc = (a0 + a1) + (a2 + a3)
            out_v[pl.ds(cbase + g * L, L)] = 1.0 / (1.0 + jnp.exp(-acc))
            return carry

        lax.fori_loop(0, CHUNK // L, group_body, 0)

    def outer(o, carry):
        for b in range(N_SLOTS):
            chunk = o * N_SLOTS + b
            pltpu.make_async_copy(
                z_hbm.at[sidx_v.at[pl.ds(0, CHUNK)]], srows[b],
                sem_s[b]).wait()
            pltpu.make_async_copy(
                z_hbm.at[didx_v.at[pl.ds(0, CHUNK)]], drows[b],
                sem_d[b]).wait()
            compute(b, o, chunk)
            nxt = chunk + N_SLOTS

            @pl.when(nxt < N_ITERS)
            def _issue_next():
                issue(b, nxt)

        @pl.when(o % 5 == 4)
        def _flush():
            pltpu.sync_copy(
                out_v, out_hbm.at[pl.ds(base_w + (o // 5) * OUT_W, OUT_W)])

        return carry

    lax.fori_loop(0, N_ITERS // N_SLOTS, outer, 0)


@jax.jit
def _run(z, src, dst):
    mesh = plsc.VectorSubcoreMesh(
        core_axis_name="c", subcore_axis_name="s",
        num_cores=NC, num_subcores=NS)
    kfn = pl.kernel(
        _sc_body,
        out_type=jax.ShapeDtypeStruct((E,), jnp.float32),
        mesh=mesh,
        scratch_types=[
            pltpu.VMEM((PER_W,), jnp.int32),
            pltpu.VMEM((PER_W,), jnp.int32),
            [pltpu.VMEM((CHUNK, D), jnp.float32) for _ in range(N_SLOTS)],
            [pltpu.VMEM((CHUNK, D), jnp.float32) for _ in range(N_SLOTS)],
            pltpu.VMEM((OUT_W,), jnp.float32),
            pltpu.SemaphoreType.DMA,
            pltpu.SemaphoreType.DMA,
            [pltpu.SemaphoreType.DMA for _ in range(N_SLOTS)],
            [pltpu.SemaphoreType.DMA for _ in range(N_SLOTS)],
        ],
        compiler_params=pltpu.CompilerParams(needs_layout_passes=False),
    )
    return kfn(z, src, dst)


def kernel(z, edge_index):
    src = edge_index[0].astype(jnp.int32)
    dst = edge_index[1].astype(jnp.int32)
    return _run(z, src, dst)


# speedup vs baseline: 11.0031x; 7.1817x over previous
import functools

import jax
import jax.numpy as jnp
from jax import lax
from jax.experimental import pallas as pl
from jax.experimental.pallas import tpu as pltpu
from jax.experimental.pallas import tpu_sc as plsc

NC = 2
NS = 16
L = 16
NW = NC * NS

E = 320000
D = 128
PER_W = E // NW
CHUNK = 80
N_ITERS = PER_W // CHUNK
N_SLOTS = 5
OUT_W = 5 * N_SLOTS * CHUNK


def _sc_body(z_hbm, src_hbm, dst_hbm, out_hbm, sidx_v, didx_v,
             srows, drows, out_v, sem_i0, sem_i1, sem_s, sem_d):
    wid = lax.axis_index("s") * NC + lax.axis_index("c")
    lane = lax.iota(jnp.int32, L)
    base_w = wid * PER_W

    ci0 = pltpu.async_copy(src_hbm.at[pl.ds(base_w, PER_W)], sidx_v, sem_i0)
    ci1 = pltpu.async_copy(dst_hbm.at[pl.ds(base_w, PER_W)], didx_v, sem_i1)
    ci0.wait()
    ci1.wait()

    def issue(b, chunk):
        off = chunk * CHUNK
        pltpu.async_copy(
            z_hbm.at[sidx_v.at[pl.ds(off, CHUNK)]], srows[b], sem_s[b])
        pltpu.async_copy(
            z_hbm.at[didx_v.at[pl.ds(off, CHUNK)]], drows[b], sem_d[b])

    for b in range(N_SLOTS):
        issue(b, b)

    def compute(b, o, chunk):
        cbase = ((o % 5) * N_SLOTS + (chunk - o * N_SLOTS)) * CHUNK

        def group_body(g, carry):
            eids = g * L + lane

            def d_blk(j, accs):
                a0, a1, a2, a3 = accs
                prods = []
                for u in range(8):
                    dv = (lane + (j * 8 + u)) & (D - 1)
                    s = plsc.load_gather(srows[b], [eids, dv])
                    t = plsc.load_gather(drows[b], [eids, dv])
                    prods.append(s * t)
                a0 = a0 + (prods[0] + prods[1])
                a1 = a1 + (prods[2] + prods[3])
                a2 = a2 + (prods[4] + prods[5])
                a3 = a3 + (prods[6] + prods[7])
                return a0, a1, a2, a3

            z4 = jnp.zeros((L,), jnp.float32)
            a0, a1, a2, a3 = lax.fori_loop(0, D // 8, d_blk,
                                           (z4, z4, z4, z4))
            acc = (a0 + a1) + (a2 + a3)
            out_v[pl.ds(cbase + g * L, L)] = 1.0 / (1.0 + jnp.exp(-acc))
            return carry

        lax.fori_loop(0, CHUNK // L, group_body, 0)

    def outer(o, carry):
        for b in range(N_SLOTS):
            chunk = o * N_SLOTS + b
            pltpu.make_async_copy(
                z_hbm.at[sidx_v.at[pl.ds(0, CHUNK)]], srows[b],
                sem_s[b]).wait()
            pltpu.make_async_copy(
                z_hbm.at[didx_v.at[pl.ds(0, CHUNK)]], drows[b],
                sem_d[b]).wait()
            compute(b, o, chunk)
            nxt = chunk + N_SLOTS

            @pl.when(nxt < N_ITERS)
            def _issue_next():
                issue(b, nxt)

        @pl.when(o % 5 == 4)
        def _flush():
            pltpu.sync_copy(
                out_v, out_hbm.at[pl.ds(base_w + (o // 5) * OUT_W, OUT_W)])

        return carry

    lax.fori_loop(0, N_ITERS // N_SLOTS, outer, 0)


@jax.jit
def _run(z, src, dst):
    mesh = plsc.VectorSubcoreMesh(
        core_axis_name="c", subcore_axis_name="s",
        num_cores=NC, num_subcores=NS)
    kfn = pl.kernel(
        _sc_body,
        out_type=jax.ShapeDtypeStruct((E,), jnp.float32),
        mesh=mesh,
        scratch_types=[
            pltpu.VMEM((PER_W,), jnp.int32),
            pltpu.VMEM((PER_W,), jnp.int32),
            [pltpu.VMEM((CHUNK, D), jnp.float32) for _ in range(N_SLOTS)],
            [pltpu.VMEM((CHUNK, D), jnp.float32) for _ in range(N_SLOTS)],
            pltpu.VMEM((OUT_W,), jnp.float32),
            pltpu.SemaphoreType.DMA,
            pltpu.SemaphoreType.DMA,
            [pltpu.SemaphoreType.DMA for _ in range(N_SLOTS)],
            [pltpu.SemaphoreType.DMA for _ in range(N_SLOTS)],
        ],
        compiler_params=pltpu.CompilerParams(needs_layout_passes=False),
    )
    return kfn(z, src, dst)


def kernel(z, edge_index):
    src = edge_index[0].astype(jnp.int32)
    dst = edge_index[1].astype(jnp.int32)
    return _run(z, src, dst)


# speedup vs baseline: 11.0430x; 1.0036x over previous
import functools

import jax
import jax.numpy as jnp
from jax import lax
from jax.experimental import pallas as pl
from jax.experimental.pallas import tpu as pltpu
from jax.experimental.pallas import tpu_sc as plsc

NC = 2
NS = 16
L = 16
NW = NC * NS

E = 320000
D = 128
PER_W = E // NW
CHUNK = 80
N_ITERS = PER_W // CHUNK
N_SLOTS = 5
OUT_W = 5 * N_SLOTS * CHUNK


def _sc_body(z_hbm, src_hbm, dst_hbm, out_hbm, sidx_v, didx_v,
             srows, drows, out_v, sem_i0, sem_i1, sem_s, sem_d):
    wid = lax.axis_index("s") * NC + lax.axis_index("c")
    lane = lax.iota(jnp.int32, L)
    base_w = wid * PER_W

    ci0 = pltpu.async_copy(src_hbm.at[pl.ds(base_w, PER_W)], sidx_v, sem_i0)
    ci1 = pltpu.async_copy(dst_hbm.at[pl.ds(base_w, PER_W)], didx_v, sem_i1)
    ci0.wait()
    ci1.wait()

    def issue(b, chunk):
        off = chunk * CHUNK
        pltpu.async_copy(
            z_hbm.at[sidx_v.at[pl.ds(off, CHUNK)]], srows[b], sem_s[b])
        pltpu.async_copy(
            z_hbm.at[didx_v.at[pl.ds(off, CHUNK)]], drows[b], sem_d[b])

    for b in range(N_SLOTS):
        issue(b, b)

    def compute(b, o, chunk):
        cbase = ((o % 5) * N_SLOTS + (chunk - o * N_SLOTS)) * CHUNK

        def group_body(g, carry):
            eids = g * L + lane

            def d_blk(j, accs):
                a0, a1, a2, a3 = accs
                prods = []
                for u in range(16):
                    dv = (lane + (j * 16 + u)) & (D - 1)
                    s = plsc.load_gather(srows[b], [eids, dv])
                    t = plsc.load_gather(drows[b], [eids, dv])
                    prods.append(s * t)
                a0 = a0 + ((prods[0] + prods[1]) + (prods[2] + prods[3]))
                a1 = a1 + ((prods[4] + prods[5]) + (prods[6] + prods[7]))
                a2 = a2 + ((prods[8] + prods[9]) + (prods[10] + prods[11]))
                a3 = a3 + ((prods[12] + prods[13]) + (prods[14] + prods[15]))
                return a0, a1, a2, a3

            z4 = jnp.zeros((L,), jnp.float32)
            a0, a1, a2, a3 = lax.fori_loop(0, D // 16, d_blk,
                                           (z4, z4, z4, z4))
            acc = (a0 + a1) + (a2 + a3)
            out_v[pl.ds(cbase + g * L, L)] = 1.0 / (1.0 + jnp.exp(-acc))
            return carry

        lax.fori_loop(0, CHUNK // L, group_body, 0)

    def outer(o, carry):
        for b in range(N_SLOTS):
            chunk = o * N_SLOTS + b
            pltpu.make_async_copy(
                z_hbm.at[sidx_v.at[pl.ds(0, CHUNK)]], srows[b],
                sem_s[b]).wait()
            pltpu.make_async_copy(
                z_hbm.at[didx_v.at[pl.ds(0, CHUNK)]], drows[b],
                sem_d[b]).wait()
            compute(b, o, chunk)
            nxt = chunk + N_SLOTS

            @pl.when(nxt < N_ITERS)
            def _issue_next():
                issue(b, nxt)

        @pl.when(o % 5 == 4)
        def _flush():
            pltpu.sync_copy(
                out_v, out_hbm.at[pl.ds(base_w + (o // 5) * OUT_W, OUT_W)])

        return carry

    lax.fori_loop(0, N_ITERS // N_SLOTS, outer, 0)


@jax.jit
def _run(z, src, dst):
    mesh = plsc.VectorSubcoreMesh(
        core_axis_name="c", subcore_axis_name="s",
        num_cores=NC, num_subcores=NS)
    kfn = pl.kernel(
        _sc_body,
        out_type=jax.ShapeDtypeStruct((E,), jnp.float32),
        mesh=mesh,
        scratch_types=[
            pltpu.VMEM((PER_W,), jnp.int32),
            pltpu.VMEM((PER_W,), jnp.int32),
            [pltpu.VMEM((CHUNK, D), jnp.float32) for _ in range(N_SLOTS)],
            [pltpu.VMEM((CHUNK, D), jnp.float32) for _ in range(N_SLOTS)],
            pltpu.VMEM((OUT_W,), jnp.float32),
            pltpu.SemaphoreType.DMA,
            pltpu.SemaphoreType.DMA,
            [pltpu.SemaphoreType.DMA for _ in range(N_SLOTS)],
            [pltpu.SemaphoreType.DMA for _ in range(N_SLOTS)],
        ],
        compiler_params=pltpu.CompilerParams(needs_layout_passes=False),
    )
    return kfn(z, src, dst)


def kernel(z, edge_index):
    src = edge_index[0].astype(jnp.int32)
    dst = edge_index[1].astype(jnp.int32)
    return _run(z, src, dst)
